# baseline (speedup 1.0000x reference)
import functools
import math

import jax
import jax.numpy as jnp
from jax import lax
from jax.experimental import pallas as pl
from jax.experimental.pallas import tpu as pltpu
from jax.experimental.pallas import tpu_sc as plsc

D_MODEL = 64
SCALE = math.sqrt(D_MODEL)
LANES = 16
CHUNK = 128
NUM_CORES = 2
NUM_SUBCORES = 16
NUM_WORKERS = NUM_CORES * NUM_SUBCORES


@functools.lru_cache(maxsize=None)
def _make_kernel(batch: int, hist: int):
    B = batch * hist
    assert B % (NUM_WORKERS * CHUNK) == 0 and batch % CHUNK == 0
    blocks_per_h = batch // CHUNK
    b_per_w = B // NUM_WORKERS
    nch = b_per_w // CHUNK
    mesh = plsc.VectorSubcoreMesh(core_axis_name="c", subcore_axis_name="s")

    @functools.partial(
        pl.kernel,
        mesh=mesh,
        out_type=jax.ShapeDtypeStruct((hist, D_MODEL, batch), jnp.float32),
        scratch_types=[
            pltpu.VMEM((CHUNK,), jnp.int32),
            pltpu.VMEM((CHUNK,), jnp.int32),
            pltpu.VMEM((CHUNK, D_MODEL), jnp.float32),
            pltpu.VMEM((CHUNK, D_MODEL), jnp.float32),
            pltpu.VMEM((D_MODEL, CHUNK), jnp.float32),
            pltpu.SemaphoreType.DMA,
            pltpu.SemaphoreType.DMA,
            pltpu.SemaphoreType.DMA,
            pltpu.SemaphoreType.DMA,
        ],
        compiler_params=pltpu.CompilerParams(
            use_tc_tiling_on_sc=False, needs_layout_passes=False
        ),
    )
    def k(xt_hbm, table_hbm, out_hbm, i0, i1, b0_, b1_, tbuf, si0, si1, sg0, sg1):
        wid = lax.axis_index("s") * NUM_CORES + lax.axis_index("c")
        cbase = wid * nch

        idxs = (i0, i1)
        bufs = (b0_, b1_)
        isems = (si0, si1)
        gsems = (sg0, sg1)

        def fetch_idx(j, u):
            pltpu.async_copy(
                xt_hbm.at[pl.ds((cbase + j) * CHUNK, CHUNK)], idxs[u], isems[u]
            )

        def wait_idx(j, u):
            pltpu.make_async_copy(
                xt_hbm.at[pl.ds((cbase + j) * CHUNK, CHUNK)], idxs[u], isems[u]
            ).wait()

        def fire_gather(u):
            pltpu.async_copy(table_hbm.at[idxs[u]], bufs[u], gsems[u])

        def wait_gather(u):
            pltpu.make_async_copy(table_hbm.at[idxs[u]], bufs[u], gsems[u]).wait()

        rows = [
            (lax.iota(jnp.int32, LANES) + c * LANES) for c in range(D_MODEL // LANES)
        ]

        fetch_idx(0, 0)
        wait_idx(0, 0)
        fire_gather(0)
        fetch_idx(1, 1)

        def outer(g, _):
            for u in range(2):
                j = g * 2 + u
                o = 1 - u
                gc = cbase + j
                h = gc // blocks_per_h
                bb0 = (gc % blocks_per_h) * CHUNK

                @pl.when(j + 1 < nch)
                def _():
                    wait_idx(j + 1, o)
                    fire_gather(o)

                wait_gather(u)

                @pl.when(j + 2 < nch)
                def _():
                    fetch_idx(j + 2, u)

                buf = bufs[u]

                def trans_row(r, _):
                    col = jnp.full((LANES,), r, jnp.int32)
                    for c in range(D_MODEL // LANES):
                        v = buf[r, pl.ds(c * LANES, LANES)] * SCALE
                        plsc.store_scatter(tbuf, [rows[c], col], v)
                    return 0

                lax.fori_loop(0, CHUNK, trans_row, 0)
                pltpu.sync_copy(tbuf, out_hbm.at[h, :, pl.ds(bb0, CHUNK)])
            return 0

        lax.fori_loop(0, nch // 2, outer, 0)

    return k


def kernel(x, table):
    batch, hist = x.shape
    xt = jnp.transpose(x).reshape(-1).astype(jnp.int32)
    out = _make_kernel(batch, hist)(xt, table)
    return jnp.transpose(out, (2, 0, 1))


# speedup vs baseline: 1.4540x; 1.4540x over previous
import functools
import math

import jax
import jax.numpy as jnp
from jax import lax
from jax.experimental import pallas as pl
from jax.experimental.pallas import tpu as pltpu
from jax.experimental.pallas import tpu_sc as plsc

D_MODEL = 64
SCALE = math.sqrt(D_MODEL)
LANES = 16
CHUNK = 128
NUM_CORES = 2
NUM_SUBCORES = 16
NUM_WORKERS = NUM_CORES * NUM_SUBCORES


@functools.lru_cache(maxsize=None)
def _make_kernel(batch: int, hist: int):
    B = batch * hist
    assert B % (NUM_WORKERS * CHUNK) == 0 and batch % CHUNK == 0
    blocks_per_h = batch // CHUNK
    b_per_w = B // NUM_WORKERS
    nch = b_per_w // CHUNK
    mesh = plsc.VectorSubcoreMesh(core_axis_name="c", subcore_axis_name="s")

    @functools.partial(
        pl.kernel,
        mesh=mesh,
        out_type=jax.ShapeDtypeStruct((hist, D_MODEL, batch), jnp.float32),
        scratch_types=[
            pltpu.VMEM((CHUNK,), jnp.int32),
            pltpu.VMEM((CHUNK,), jnp.int32),
            pltpu.VMEM((CHUNK, D_MODEL), jnp.float32),
            pltpu.VMEM((CHUNK, D_MODEL), jnp.float32),
            pltpu.VMEM((D_MODEL, CHUNK + 1), jnp.float32),
            pltpu.SemaphoreType.DMA,
            pltpu.SemaphoreType.DMA,
            pltpu.SemaphoreType.DMA,
            pltpu.SemaphoreType.DMA,
        ],
        compiler_params=pltpu.CompilerParams(
            use_tc_tiling_on_sc=False, needs_layout_passes=False
        ),
    )
    def k(xt_hbm, table_hbm, out_hbm, i0, i1, b0_, b1_, tbuf, si0, si1, sg0, sg1):
        wid = lax.axis_index("s") * NUM_CORES + lax.axis_index("c")
        cbase = wid * nch

        idxs = (i0, i1)
        bufs = (b0_, b1_)
        isems = (si0, si1)
        gsems = (sg0, sg1)

        def fetch_idx(j, u):
            pltpu.async_copy(
                xt_hbm.at[pl.ds((cbase + j) * CHUNK, CHUNK)], idxs[u], isems[u]
            )

        def wait_idx(j, u):
            pltpu.make_async_copy(
                xt_hbm.at[pl.ds((cbase + j) * CHUNK, CHUNK)], idxs[u], isems[u]
            ).wait()

        def fire_gather(u):
            pltpu.async_copy(table_hbm.at[idxs[u]], bufs[u], gsems[u])

        def wait_gather(u):
            pltpu.make_async_copy(table_hbm.at[idxs[u]], bufs[u], gsems[u]).wait()

        rows = [
            (lax.iota(jnp.int32, LANES) + c * LANES) for c in range(D_MODEL // LANES)
        ]

        fetch_idx(0, 0)
        wait_idx(0, 0)
        fire_gather(0)
        fetch_idx(1, 1)

        def outer(g, _):
            for u in range(2):
                j = g * 2 + u
                o = 1 - u
                gc = cbase + j
                h = gc // blocks_per_h
                bb0 = (gc % blocks_per_h) * CHUNK

                @pl.when(j + 1 < nch)
                def _():
                    wait_idx(j + 1, o)
                    fire_gather(o)

                wait_gather(u)

                @pl.when(j + 2 < nch)
                def _():
                    fetch_idx(j + 2, u)

                buf = bufs[u]

                def trans_row(r, _):
                    col = jnp.full((LANES,), r, jnp.int32)
                    for c in range(D_MODEL // LANES):
                        v = buf[r, pl.ds(c * LANES, LANES)] * SCALE
                        plsc.store_scatter(tbuf, [rows[c], col], v)
                    return 0

                lax.fori_loop(0, CHUNK, trans_row, 0)
                pltpu.sync_copy(tbuf.at[:, pl.ds(0, CHUNK)], out_hbm.at[h, :, pl.ds(bb0, CHUNK)])
            return 0

        lax.fori_loop(0, nch // 2, outer, 0)

    return k


def kernel(x, table):
    batch, hist = x.shape
    xt = jnp.transpose(x).reshape(-1).astype(jnp.int32)
    out = _make_kernel(batch, hist)(xt, table)
    return jnp.transpose(out, (2, 0, 1))


# speedup vs baseline: 1.4847x; 1.0211x over previous
import functools
import math

import jax
import jax.numpy as jnp
from jax import lax
from jax.experimental import pallas as pl
from jax.experimental.pallas import tpu as pltpu
from jax.experimental.pallas import tpu_sc as plsc

D_MODEL = 64
SCALE = math.sqrt(D_MODEL)
LANES = 16
CHUNK = 128
NUM_CORES = 2
NUM_SUBCORES = 16
NUM_WORKERS = NUM_CORES * NUM_SUBCORES


@functools.lru_cache(maxsize=None)
def _make_kernel(batch: int, hist: int):
    B = batch * hist
    assert B % (NUM_WORKERS * CHUNK) == 0 and batch % CHUNK == 0
    blocks_per_h = batch // CHUNK
    b_per_w = B // NUM_WORKERS
    nch = b_per_w // CHUNK
    mesh = plsc.VectorSubcoreMesh(core_axis_name="c", subcore_axis_name="s")

    @functools.partial(
        pl.kernel,
        mesh=mesh,
        out_type=jax.ShapeDtypeStruct((hist, D_MODEL, batch), jnp.float32),
        scratch_types=[
            pltpu.VMEM((CHUNK,), jnp.int32),
            pltpu.VMEM((CHUNK,), jnp.int32),
            pltpu.VMEM((CHUNK, D_MODEL), jnp.float32),
            pltpu.VMEM((CHUNK, D_MODEL), jnp.float32),
            pltpu.VMEM((D_MODEL, CHUNK + 1), jnp.float32),
            pltpu.VMEM((D_MODEL, CHUNK + 1), jnp.float32),
            pltpu.SemaphoreType.DMA,
            pltpu.SemaphoreType.DMA,
            pltpu.SemaphoreType.DMA,
            pltpu.SemaphoreType.DMA,
            pltpu.SemaphoreType.DMA,
            pltpu.SemaphoreType.DMA,
        ],
        compiler_params=pltpu.CompilerParams(
            use_tc_tiling_on_sc=False, needs_layout_passes=False
        ),
    )
    def k(xt_hbm, table_hbm, out_hbm, i0, i1, b0_, b1_, t0, t1, si0, si1, sg0, sg1, so0, so1):
        wid = lax.axis_index("s") * NUM_CORES + lax.axis_index("c")
        cbase = wid * nch

        idxs = (i0, i1)
        bufs = (b0_, b1_)
        tbufs = (t0, t1)
        isems = (si0, si1)
        gsems = (sg0, sg1)
        osems = (so0, so1)

        def fetch_idx(j, u):
            pltpu.async_copy(
                xt_hbm.at[pl.ds((cbase + j) * CHUNK, CHUNK)], idxs[u], isems[u]
            )

        def wait_idx(j, u):
            pltpu.make_async_copy(
                xt_hbm.at[pl.ds((cbase + j) * CHUNK, CHUNK)], idxs[u], isems[u]
            ).wait()

        def fire_gather(u):
            pltpu.async_copy(table_hbm.at[idxs[u]], bufs[u], gsems[u])

        def wait_gather(u):
            pltpu.make_async_copy(table_hbm.at[idxs[u]], bufs[u], gsems[u]).wait()

        rows = [
            (lax.iota(jnp.int32, LANES) + c * LANES) for c in range(D_MODEL // LANES)
        ]

        fetch_idx(0, 0)
        wait_idx(0, 0)
        fire_gather(0)
        fetch_idx(1, 1)

        def outer(g, _):
            for u in range(2):
                j = g * 2 + u
                o = 1 - u
                gc = cbase + j
                h = gc // blocks_per_h
                bb0 = (gc % blocks_per_h) * CHUNK

                @pl.when(j + 1 < nch)
                def _():
                    wait_idx(j + 1, o)
                    fire_gather(o)

                wait_gather(u)

                @pl.when(j + 2 < nch)
                def _():
                    fetch_idx(j + 2, u)

                buf = bufs[u]
                tbuf = tbufs[u]

                @pl.when(j >= 2)
                def _():
                    ph = (gc - 2) // blocks_per_h
                    pb = ((gc - 2) % blocks_per_h) * CHUNK
                    pltpu.make_async_copy(
                        tbuf.at[:, pl.ds(0, CHUNK)],
                        out_hbm.at[ph, :, pl.ds(pb, CHUNK)],
                        osems[u],
                    ).wait()

                def trans_row(r, _):
                    col = jnp.full((LANES,), r, jnp.int32)
                    for c in range(D_MODEL // LANES):
                        v = buf[r, pl.ds(c * LANES, LANES)] * SCALE
                        plsc.store_scatter(tbuf, [rows[c], col], v)
                    return 0

                lax.fori_loop(0, CHUNK, trans_row, 0)
                pltpu.async_copy(
                    tbuf.at[:, pl.ds(0, CHUNK)],
                    out_hbm.at[h, :, pl.ds(bb0, CHUNK)],
                    osems[u],
                )
            return 0

        lax.fori_loop(0, nch // 2, outer, 0)

        for u in range(2):
            j = nch - 2 + u
            gc = cbase + j
            h = gc // blocks_per_h
            bb0 = (gc % blocks_per_h) * CHUNK
            pltpu.make_async_copy(
                tbufs[u].at[:, pl.ds(0, CHUNK)],
                out_hbm.at[h, :, pl.ds(bb0, CHUNK)],
                osems[u],
            ).wait()

    return k


def kernel(x, table):
    batch, hist = x.shape
    xt = jnp.transpose(x).reshape(-1).astype(jnp.int32)
    out = _make_kernel(batch, hist)(xt, table)
    return jnp.transpose(out, (2, 0, 1))


# speedup vs baseline: 1.9900x; 1.3403x over previous
import functools
import math

import jax
import jax.numpy as jnp
from jax import lax
from jax.experimental import pallas as pl
from jax.experimental.pallas import tpu as pltpu
from jax.experimental.pallas import tpu_sc as plsc

D_MODEL = 64
SCALE = math.sqrt(D_MODEL)
LANES = 16
CHUNK = 128
NUM_CORES = 2
NUM_SUBCORES = 16
NUM_WORKERS = NUM_CORES * NUM_SUBCORES


@functools.lru_cache(maxsize=None)
def _make_kernel(batch: int, hist: int):
    B = batch * hist
    assert B % (NUM_WORKERS * CHUNK) == 0 and batch % CHUNK == 0
    blocks_per_h = batch // CHUNK
    b_per_w = B // NUM_WORKERS
    nch = b_per_w // CHUNK
    mesh = plsc.VectorSubcoreMesh(core_axis_name="c", subcore_axis_name="s")

    @functools.partial(
        pl.kernel,
        mesh=mesh,
        out_type=jax.ShapeDtypeStruct((hist, D_MODEL, batch), jnp.float32),
        scratch_types=[
            pltpu.VMEM((CHUNK,), jnp.int32),
            pltpu.VMEM((CHUNK,), jnp.int32),
            pltpu.VMEM((CHUNK, D_MODEL), jnp.float32),
            pltpu.VMEM((CHUNK, D_MODEL), jnp.float32),
            pltpu.VMEM((D_MODEL, CHUNK + 1), jnp.float32),
            pltpu.VMEM((D_MODEL, CHUNK + 1), jnp.float32),
            pltpu.SemaphoreType.DMA,
            pltpu.SemaphoreType.DMA,
            pltpu.SemaphoreType.DMA,
            pltpu.SemaphoreType.DMA,
            pltpu.SemaphoreType.DMA,
            pltpu.SemaphoreType.DMA,
        ],
        compiler_params=pltpu.CompilerParams(
            use_tc_tiling_on_sc=False, needs_layout_passes=False
        ),
    )
    def k(xt_hbm, table_hbm, out_hbm, i0, i1, b0_, b1_, t0, t1, si0, si1, sg0, sg1, so0, so1):
        wid = lax.axis_index("s") * NUM_CORES + lax.axis_index("c")
        cbase = wid * nch

        idxs = (i0, i1)
        bufs = (b0_, b1_)
        tbufs = (t0, t1)
        isems = (si0, si1)
        gsems = (sg0, sg1)
        osems = (so0, so1)

        def fetch_idx(j, u):
            pltpu.async_copy(
                xt_hbm.at[pl.ds((cbase + j) * CHUNK, CHUNK)], idxs[u], isems[u]
            )

        def wait_idx(j, u):
            pltpu.make_async_copy(
                xt_hbm.at[pl.ds((cbase + j) * CHUNK, CHUNK)], idxs[u], isems[u]
            ).wait()

        def fire_gather(u):
            pltpu.async_copy(table_hbm.at[idxs[u]], bufs[u], gsems[u])

        def wait_gather(u):
            pltpu.make_async_copy(table_hbm.at[idxs[u]], bufs[u], gsems[u]).wait()

        rows = [
            (lax.iota(jnp.int32, LANES) + c * LANES) for c in range(D_MODEL // LANES)
        ]

        fetch_idx(0, 0)
        wait_idx(0, 0)
        fire_gather(0)
        fetch_idx(1, 1)

        def outer(g, _):
            for u in range(2):
                j = g * 2 + u
                o = 1 - u
                gc = cbase + j
                h = gc // blocks_per_h
                bb0 = (gc % blocks_per_h) * CHUNK

                @pl.when(j + 1 < nch)
                def _():
                    wait_idx(j + 1, o)
                    fire_gather(o)

                wait_gather(u)

                @pl.when(j + 2 < nch)
                def _():
                    fetch_idx(j + 2, u)

                buf = bufs[u]
                tbuf = tbufs[u]

                @pl.when(j >= 2)
                def _():
                    ph = (gc - 2) // blocks_per_h
                    pb = ((gc - 2) % blocks_per_h) * CHUNK
                    pltpu.make_async_copy(
                        tbuf.at[:, pl.ds(0, CHUNK)],
                        out_hbm.at[ph, :, pl.ds(pb, CHUNK)],
                        osems[u],
                    ).wait()

                @plsc.parallel_loop(0, CHUNK, unroll=8)
                def _(r):
                    col = jnp.full((LANES,), r, jnp.int32)
                    for c in range(D_MODEL // LANES):
                        v = buf[r, pl.ds(c * LANES, LANES)] * SCALE
                        plsc.store_scatter(tbuf, [rows[c], col], v)
                pltpu.async_copy(
                    tbuf.at[:, pl.ds(0, CHUNK)],
                    out_hbm.at[h, :, pl.ds(bb0, CHUNK)],
                    osems[u],
                )
            return 0

        lax.fori_loop(0, nch // 2, outer, 0)

        for u in range(2):
            j = nch - 2 + u
            gc = cbase + j
            h = gc // blocks_per_h
            bb0 = (gc % blocks_per_h) * CHUNK
            pltpu.make_async_copy(
                tbufs[u].at[:, pl.ds(0, CHUNK)],
                out_hbm.at[h, :, pl.ds(bb0, CHUNK)],
                osems[u],
            ).wait()

    return k


def kernel(x, table):
    batch, hist = x.shape
    xt = jnp.transpose(x).reshape(-1).astype(jnp.int32)
    out = _make_kernel(batch, hist)(xt, table)
    return jnp.transpose(out, (2, 0, 1))
